# Initial kernel scaffold; baseline (speedup 1.0000x reference)
import functools

import jax
import jax.numpy as jnp
from jax import lax
from jax.experimental import pallas as pl
from jax.experimental.pallas import tpu as pltpu
from jax.experimental.pallas import tpu_sc as plsc

_NC = 2
_NS = 16
_NW = _NC * _NS
_CK = 125


def _sc_scatter_partials(e4, col3, n_nodes):
    nw, ch, ck, de = e4.shape
    rpt = n_nodes // _NS

    mesh = plsc.VectorSubcoreMesh(core_axis_name="c", subcore_axis_name="s")

    @functools.partial(
        pl.kernel,
        out_type=jax.ShapeDtypeStruct((_NC, n_nodes, de), jnp.float32),
        mesh=mesh,
        scratch_types=[
            pltpu.VMEM((ch, ck), jnp.int32),
            pltpu.VMEM((ck, de), jnp.float32),
            pltpu.VMEM((rpt, de), jnp.float32),
            pltpu.VMEM_SHARED((n_nodes, de), jnp.float32),
        ],
    )
    def k(e_hbm, col_hbm, out_hbm, idx_v, ebuf, rowbuf, agg_sh):
        c = lax.axis_index("c")
        s = lax.axis_index("s")
        wid = s * _NC + c

        def zero_row(i, carry):
            rowbuf[i, :] = jnp.zeros((de,), jnp.float32)
            return carry

        lax.fori_loop(0, rpt, zero_row, 0)
        pltpu.sync_copy(rowbuf, agg_sh.at[pl.ds(s * rpt, rpt)])
        plsc.subcore_barrier()

        pltpu.sync_copy(col_hbm.at[wid], idx_v)

        def body(g, carry):
            pltpu.sync_copy(e_hbm.at[wid, g], ebuf)
            pltpu.sync_copy(ebuf, agg_sh.at[idx_v.at[g]], add=True)
            return carry

        lax.fori_loop(0, ch, body, 0)
        plsc.subcore_barrier()

        pltpu.sync_copy(agg_sh.at[pl.ds(s * rpt, rpt)], rowbuf)
        pltpu.sync_copy(rowbuf, out_hbm.at[c, pl.ds(s * rpt, rpt)])

    return k(e4, col3)


def _tc_mlp(x, parts, batch3, u, w1a, w1b, w1c, b1, w2, b2, w3, b3, bn):
    n, df = x.shape
    grid = n // bn
    de = parts.shape[2]
    dg, du = u.shape
    hh = w2.shape[0]
    t = w3.shape[1]

    def body(x_r, p_r, b_r, u_r, w1a_r, w1b_r, w1c_r, b1_r, w2_r, b2_r, w3_r,
             b3_r, o_r):
        xb = x_r[...]
        agg = p_r[0] + p_r[1]
        bblk = b_r[0, 0, :]
        oh = (bblk[:, None] == lax.broadcasted_iota(jnp.int32, (bn, dg), 1))
        oh = oh.astype(jnp.float32)
        uw = jnp.dot(u_r[...], w1c_r[...], preferred_element_type=jnp.float32)
        pre = (jnp.dot(xb, w1a_r[...], preferred_element_type=jnp.float32)
               + jnp.dot(agg, w1b_r[...], preferred_element_type=jnp.float32)
               + jnp.dot(oh, uw, preferred_element_type=jnp.float32)
               + b1_r[...])
        h1 = jnp.where(pre > 0, pre, 0.01 * pre)
        pre2 = jnp.dot(h1, w2_r[...], preferred_element_type=jnp.float32) + b2_r[...]
        h2 = jnp.where(pre2 > 0, pre2, 0.01 * pre2)
        o_r[...] = jnp.dot(h2, w3_r[...], preferred_element_type=jnp.float32) + b3_r[...]

    return pl.pallas_call(
        body,
        grid=(grid,),
        in_specs=[
            pl.BlockSpec((bn, df), lambda i: (i, 0)),
            pl.BlockSpec((2, bn, de), lambda i: (0, i, 0)),
            pl.BlockSpec((1, 1, bn), lambda i: (i, 0, 0)),
            pl.BlockSpec((dg, du), lambda i: (0, 0)),
            pl.BlockSpec((df, hh), lambda i: (0, 0)),
            pl.BlockSpec((de, hh), lambda i: (0, 0)),
            pl.BlockSpec((du, hh), lambda i: (0, 0)),
            pl.BlockSpec((1, hh), lambda i: (0, 0)),
            pl.BlockSpec((hh, hh), lambda i: (0, 0)),
            pl.BlockSpec((1, hh), lambda i: (0, 0)),
            pl.BlockSpec((hh, t), lambda i: (0, 0)),
            pl.BlockSpec((1, t), lambda i: (0, 0)),
        ],
        out_specs=pl.BlockSpec((bn, t), lambda i: (i, 0)),
        out_shape=jax.ShapeDtypeStruct((n, t), jnp.float32),
    )(x, parts, batch3, u, w1a, w1b, w1c, b1, w2, b2, w3, b3)


def kernel(x, edge_index, edge_attr, u, batch, W1, b1, W2, b2, W3, b3):
    n, df = x.shape
    e, de = edge_attr.shape
    ept = e // _NW
    ch = ept // _CK

    col = edge_index[1]
    e4 = edge_attr.reshape(_NW, ch, _CK, de)
    col3 = col.reshape(_NW, ch, _CK)
    parts = _sc_scatter_partials(e4, col3, n)

    w1a = W1[:df]
    w1b = W1[df:df + de]
    w1c = W1[df + de:]
    bn = 500
    batch3 = batch.reshape(n // bn, 1, bn)
    return _tc_mlp(x, parts, batch3, u, w1a, w1b, w1c,
                   b1.reshape(1, -1), W2, b2.reshape(1, -1), W3,
                   b3.reshape(1, -1), bn)


# speedup vs baseline: 3.8878x; 3.8878x over previous
import functools

import jax
import jax.numpy as jnp
from jax import lax
from jax.experimental import pallas as pl
from jax.experimental.pallas import tpu as pltpu
from jax.experimental.pallas import tpu_sc as plsc

_NC = 2
_NS = 16
_NW = _NC * _NS
_CK = 125


def _sc_scatter_partials(e4, col3, n_pad):
    nw, ch, ck, de = e4.shape
    rpt = n_pad // _NS

    mesh = plsc.VectorSubcoreMesh(core_axis_name="c", subcore_axis_name="s",
                                  num_cores=_NC, num_subcores=_NS)

    @functools.partial(
        pl.kernel,
        out_type=jax.ShapeDtypeStruct((_NC, n_pad, de), jnp.float32),
        mesh=mesh,
        scratch_types=[
            pltpu.VMEM((ch, ck), jnp.int32),
            pltpu.VMEM((ck, de), jnp.float32),
            pltpu.VMEM((rpt, de), jnp.float32),
            pltpu.VMEM_SHARED((n_pad, de), jnp.float32),
        ],
        compiler_params=pltpu.CompilerParams(use_tc_tiling_on_sc=False),
    )
    def k(e_hbm, col_hbm, out_hbm, idx_v, ebuf, rowbuf, agg_sh):
        c = lax.axis_index("c")
        s = lax.axis_index("s")
        wid = s * _NC + c

        @pl.loop(0, rpt)
        def zero_row(i):
            rowbuf[i, :] = jnp.zeros((de,), jnp.float32)

        pltpu.sync_copy(rowbuf, agg_sh.at[pl.ds(s * rpt, rpt)])
        plsc.subcore_barrier()

        pltpu.sync_copy(col_hbm.at[wid], idx_v)

        @pl.loop(0, ch)
        def body(g):
            pltpu.sync_copy(e_hbm.at[wid, g], ebuf)
            pltpu.sync_copy(ebuf, agg_sh.at[idx_v.at[g]], add=True)

        plsc.subcore_barrier()

        pltpu.sync_copy(agg_sh.at[pl.ds(s * rpt, rpt)], rowbuf)
        pltpu.sync_copy(rowbuf, out_hbm.at[c, pl.ds(s * rpt, rpt)])

    return k(e4, col3)


def _tc_mlp(x, parts, batch3, u, w1a, w1b, w1c, b1, w2, b2, w3, b3, bn):
    n, df = x.shape
    grid = n // bn
    de = parts.shape[2]
    dg, du = u.shape
    hh = w2.shape[0]
    t = w3.shape[1]

    def body(x_r, p_r, b_r, u_r, w1a_r, w1b_r, w1c_r, b1_r, w2_r, b2_r, w3_r,
             b3_r, o_r):
        xb = x_r[...]
        agg = p_r[0] + p_r[1]
        bblk = b_r[0, 0, :]
        oh = (bblk[:, None] == lax.broadcasted_iota(jnp.int32, (bn, dg), 1))
        oh = oh.astype(jnp.float32)
        uw = jnp.dot(u_r[...], w1c_r[...], preferred_element_type=jnp.float32)
        pre = (jnp.dot(xb, w1a_r[...], preferred_element_type=jnp.float32)
               + jnp.dot(agg, w1b_r[...], preferred_element_type=jnp.float32)
               + jnp.dot(oh, uw, preferred_element_type=jnp.float32)
               + b1_r[...])
        h1 = jnp.where(pre > 0, pre, 0.01 * pre)
        pre2 = jnp.dot(h1, w2_r[...], preferred_element_type=jnp.float32) + b2_r[...]
        h2 = jnp.where(pre2 > 0, pre2, 0.01 * pre2)
        o_r[...] = jnp.dot(h2, w3_r[...], preferred_element_type=jnp.float32) + b3_r[...]

    return pl.pallas_call(
        body,
        grid=(grid,),
        in_specs=[
            pl.BlockSpec((bn, df), lambda i: (i, 0)),
            pl.BlockSpec((2, bn, de), lambda i: (0, i, 0)),
            pl.BlockSpec((1, 1, bn), lambda i: (i, 0, 0)),
            pl.BlockSpec((dg, du), lambda i: (0, 0)),
            pl.BlockSpec((df, hh), lambda i: (0, 0)),
            pl.BlockSpec((de, hh), lambda i: (0, 0)),
            pl.BlockSpec((du, hh), lambda i: (0, 0)),
            pl.BlockSpec((1, hh), lambda i: (0, 0)),
            pl.BlockSpec((hh, hh), lambda i: (0, 0)),
            pl.BlockSpec((1, hh), lambda i: (0, 0)),
            pl.BlockSpec((hh, t), lambda i: (0, 0)),
            pl.BlockSpec((1, t), lambda i: (0, 0)),
        ],
        out_specs=pl.BlockSpec((bn, t), lambda i: (i, 0)),
        out_shape=jax.ShapeDtypeStruct((n, t), jnp.float32),
    )(x, parts, batch3, u, w1a, w1b, w1c, b1, w2, b2, w3, b3)


def kernel(x, edge_index, edge_attr, u, batch, W1, b1, W2, b2, W3, b3):
    n, df = x.shape
    e, de = edge_attr.shape
    ept = e // _NW
    ch = ept // _CK

    col = edge_index[1]
    e4 = edge_attr.reshape(_NW, ch, _CK, de)
    col3 = col.reshape(_NW, ch, _CK)
    n_pad = ((n + 8 * _NS - 1) // (8 * _NS)) * (8 * _NS)
    parts = _sc_scatter_partials(e4, col3, n_pad)

    w1a = W1[:df]
    w1b = W1[df:df + de]
    w1c = W1[df + de:]
    bn = 2000
    batch3 = batch.reshape(n // bn, 1, bn)
    return _tc_mlp(x, parts, batch3, u, w1a, w1b, w1c,
                   b1.reshape(1, -1), W2, b2.reshape(1, -1), W3,
                   b3.reshape(1, -1), bn)


# speedup vs baseline: 4.8144x; 1.2383x over previous
import functools

import jax
import jax.numpy as jnp
from jax import lax
from jax.experimental import pallas as pl
from jax.experimental.pallas import tpu as pltpu
from jax.experimental.pallas import tpu_sc as plsc

_NC = 2
_NS = 16
_NW = _NC * _NS
_CK = 125


_BF = 8


def _sc_scatter_partials(e4, ei4, n_pad):
    nw, ch, ck, de = e4.shape
    rpt = n_pad // _NS
    nt = ch // _BF

    mesh = plsc.VectorSubcoreMesh(core_axis_name="c", subcore_axis_name="s",
                                  num_cores=_NC, num_subcores=_NS)

    @functools.partial(
        pl.kernel,
        out_type=jax.ShapeDtypeStruct((_NC, n_pad, de), jnp.float32),
        mesh=mesh,
        scratch_types=[
            pltpu.VMEM((ch, ck), jnp.int32),
            pltpu.VMEM((2, _BF, ck, de), jnp.float32),
            pltpu.VMEM((rpt, de), jnp.float32),
            pltpu.VMEM_SHARED((n_pad, de), jnp.float32),
            pltpu.SemaphoreType.DMA,
            pltpu.SemaphoreType.DMA,
            pltpu.SemaphoreType.DMA,
        ],
        compiler_params=pltpu.CompilerParams(use_tc_tiling_on_sc=False),
    )
    def k(e_hbm, ei_hbm, out_hbm, idx_v, ebuf, rowbuf, agg_sh, gsem0, gsem1,
          ssem):
        c = lax.axis_index("c")
        s = lax.axis_index("s")
        wid = s * _NC + c
        gsems = [gsem0, gsem1]

        @pl.loop(0, rpt)
        def zero_row(i):
            rowbuf[i, :] = jnp.zeros((de,), jnp.float32)

        pltpu.sync_copy(rowbuf, agg_sh.at[pl.ds(s * rpt, rpt)])

        pltpu.sync_copy(ei_hbm.at[1, wid], idx_v)
        plsc.subcore_barrier()

        pltpu.async_copy(e_hbm.at[wid, pl.ds(0, _BF)], ebuf.at[0], gsem0)
        pltpu.async_copy(e_hbm.at[wid, pl.ds(_BF, _BF)], ebuf.at[1], gsem1)

        @pl.loop(0, nt, step=2)
        def outer(t):
            for b in range(2):
                tt = t + b
                pltpu.make_async_copy(
                    e_hbm.at[wid, pl.ds(tt * _BF, _BF)], ebuf.at[b], gsems[b]
                ).wait()
                descs = [
                    pltpu.async_copy(
                        ebuf.at[b, j], agg_sh.at[idx_v.at[tt * _BF + j]],
                        ssem, add=True)
                    for j in range(_BF)
                ]
                for d in descs:
                    d.wait()

                @pl.when(tt + 2 < nt)
                def _():
                    pltpu.async_copy(
                        e_hbm.at[wid, pl.ds((tt + 2) * _BF, _BF)],
                        ebuf.at[b], gsems[b])

        plsc.subcore_barrier()

        pltpu.sync_copy(agg_sh.at[pl.ds(s * rpt, rpt)], rowbuf)
        pltpu.sync_copy(rowbuf, out_hbm.at[c, pl.ds(s * rpt, rpt)])

    return k(e4, ei4)


def _tc_mlp(x, parts, batch3, u, w1a, w1b, w1c, b1, w2, b2, w3, b3, bn):
    n, df = x.shape
    grid = n // bn
    de = parts.shape[2]
    dg, du = u.shape
    hh = w2.shape[0]
    t = w3.shape[1]

    def body(x_r, p_r, b_r, u_r, w1a_r, w1b_r, w1c_r, b1_r, w2_r, b2_r, w3_r,
             b3_r, o_r):
        xb = x_r[...]
        agg = p_r[0] + p_r[1]
        bblk = b_r[0, 0, :]
        oh = (bblk[:, None] == lax.broadcasted_iota(jnp.int32, (bn, dg), 1))
        oh = oh.astype(jnp.float32)
        uw = jnp.dot(u_r[...], w1c_r[...], preferred_element_type=jnp.float32)
        pre = (jnp.dot(xb, w1a_r[...], preferred_element_type=jnp.float32)
               + jnp.dot(agg, w1b_r[...], preferred_element_type=jnp.float32)
               + jnp.dot(oh, uw, preferred_element_type=jnp.float32)
               + b1_r[...])
        h1 = jnp.where(pre > 0, pre, 0.01 * pre)
        pre2 = jnp.dot(h1, w2_r[...], preferred_element_type=jnp.float32) + b2_r[...]
        h2 = jnp.where(pre2 > 0, pre2, 0.01 * pre2)
        o_r[...] = jnp.dot(h2, w3_r[...], preferred_element_type=jnp.float32) + b3_r[...]

    return pl.pallas_call(
        body,
        grid=(grid,),
        in_specs=[
            pl.BlockSpec((bn, df), lambda i: (i, 0)),
            pl.BlockSpec((2, bn, de), lambda i: (0, i, 0)),
            pl.BlockSpec((1, 1, bn), lambda i: (i, 0, 0)),
            pl.BlockSpec((dg, du), lambda i: (0, 0)),
            pl.BlockSpec((df, hh), lambda i: (0, 0)),
            pl.BlockSpec((de, hh), lambda i: (0, 0)),
            pl.BlockSpec((du, hh), lambda i: (0, 0)),
            pl.BlockSpec((1, hh), lambda i: (0, 0)),
            pl.BlockSpec((hh, hh), lambda i: (0, 0)),
            pl.BlockSpec((1, hh), lambda i: (0, 0)),
            pl.BlockSpec((hh, t), lambda i: (0, 0)),
            pl.BlockSpec((1, t), lambda i: (0, 0)),
        ],
        out_specs=pl.BlockSpec((bn, t), lambda i: (i, 0)),
        out_shape=jax.ShapeDtypeStruct((n, t), jnp.float32),
    )(x, parts, batch3, u, w1a, w1b, w1c, b1, w2, b2, w3, b3)


def kernel(x, edge_index, edge_attr, u, batch, W1, b1, W2, b2, W3, b3):
    n, df = x.shape
    e, de = edge_attr.shape
    ept = e // _NW
    ch = ept // _CK

    e4 = edge_attr.reshape(_NW, ch, _CK, de)
    ei4 = edge_index.reshape(2, _NW, ch, _CK)
    n_pad = ((n + 8 * _NS - 1) // (8 * _NS)) * (8 * _NS)
    parts = _sc_scatter_partials(e4, ei4, n_pad)

    w1a = W1[:df]
    w1b = W1[df:df + de]
    w1c = W1[df + de:]
    bn = 2000
    batch3 = batch.reshape(n // bn, 1, bn)
    return _tc_mlp(x, parts, batch3, u, w1a, w1b, w1c,
                   b1.reshape(1, -1), W2, b2.reshape(1, -1), W3,
                   b3.reshape(1, -1), bn)


# speedup vs baseline: 5.4578x; 1.1336x over previous
import functools

import jax
import jax.numpy as jnp
from jax import lax
from jax.experimental import pallas as pl
from jax.experimental.pallas import tpu as pltpu
from jax.experimental.pallas import tpu_sc as plsc

_NC = 2
_NS = 16
_NW = _NC * _NS
_CK = 80


_BF = 5


def _sc_scatter_partials(e2, ei4, n_pad):
    ne, de = e2.shape
    _, nw, ch, ck = ei4.shape
    ept = ne // nw
    bf_e = _BF * ck
    nt = ch // _BF
    rpt = n_pad // _NS

    mesh = plsc.VectorSubcoreMesh(core_axis_name="c", subcore_axis_name="s",
                                  num_cores=_NC, num_subcores=_NS)

    @functools.partial(
        pl.kernel,
        out_type=jax.ShapeDtypeStruct((_NC, n_pad, de), jnp.float32),
        mesh=mesh,
        scratch_types=[
            pltpu.VMEM((ch, ck), jnp.int32),
            pltpu.VMEM((2, bf_e, de), jnp.float32),
            pltpu.VMEM((rpt, de), jnp.float32),
            pltpu.VMEM_SHARED((n_pad, de), jnp.float32),
            pltpu.SemaphoreType.DMA,
            pltpu.SemaphoreType.DMA,
            pltpu.SemaphoreType.DMA,
        ],
        compiler_params=pltpu.CompilerParams(use_tc_tiling_on_sc=False),
    )
    def k(e_hbm, ei_hbm, out_hbm, idx_v, ebuf, rowbuf, agg_sh, gsem0, gsem1,
          ssem):
        c = lax.axis_index("c")
        s = lax.axis_index("s")
        wid = s * _NC + c
        base = wid * ept
        gsems = [gsem0, gsem1]

        def gather_block(t, b):
            return pltpu.make_async_copy(
                e_hbm.at[pl.ds(base + t * bf_e, bf_e)], ebuf.at[b], gsems[b])

        def scatter_block(t, b):
            descs = [
                pltpu.async_copy(
                    ebuf.at[b, pl.ds(j * ck, ck)],
                    agg_sh.at[idx_v.at[t * _BF + j]], ssem, add=True)
                for j in range(_BF)
            ]
            for d in descs:
                d.wait()

        @pl.loop(0, rpt)
        def zero_row(i):
            rowbuf[i, :] = jnp.zeros((de,), jnp.float32)

        pltpu.sync_copy(rowbuf, agg_sh.at[pl.ds(s * rpt, rpt)])

        pltpu.sync_copy(ei_hbm.at[1, wid], idx_v)
        plsc.subcore_barrier()

        gather_block(0, 0).start()
        gather_block(1, 1).start()

        nt_even = nt - (nt % 2)

        @pl.loop(0, nt_even, step=2)
        def outer(t):
            for b in range(2):
                tt = t + b
                gather_block(tt, b).wait()
                scatter_block(tt, b)

                @pl.when(tt + 2 < nt)
                def _():
                    gather_block(tt + 2, b).start()

        if nt % 2:
            gather_block(nt - 1, 0).wait()
            scatter_block(nt - 1, 0)

        plsc.subcore_barrier()

        pltpu.sync_copy(agg_sh.at[pl.ds(s * rpt, rpt)], rowbuf)
        pltpu.sync_copy(rowbuf, out_hbm.at[c, pl.ds(s * rpt, rpt)])

    return k(e2, ei4)


def _tc_mlp(x, parts, batch3, u, w1a, w1b, w1c, b1, w2, b2, w3, b3, bn):
    n, df = x.shape
    grid = n // bn
    de = parts.shape[2]
    dg, du = u.shape
    hh = w2.shape[0]
    t = w3.shape[1]

    def body(x_r, p_r, b_r, u_r, w1a_r, w1b_r, w1c_r, b1_r, w2_r, b2_r, w3_r,
             b3_r, o_r):
        xb = x_r[...]
        agg = p_r[0] + p_r[1]
        bblk = b_r[0, 0, :]
        oh = (bblk[:, None] == lax.broadcasted_iota(jnp.int32, (bn, dg), 1))
        oh = oh.astype(jnp.float32)
        uw = jnp.dot(u_r[...], w1c_r[...], preferred_element_type=jnp.float32)
        pre = (jnp.dot(xb, w1a_r[...], preferred_element_type=jnp.float32)
               + jnp.dot(agg, w1b_r[...], preferred_element_type=jnp.float32)
               + jnp.dot(oh, uw, preferred_element_type=jnp.float32)
               + b1_r[...])
        h1 = jnp.where(pre > 0, pre, 0.01 * pre)
        pre2 = jnp.dot(h1, w2_r[...], preferred_element_type=jnp.float32) + b2_r[...]
        h2 = jnp.where(pre2 > 0, pre2, 0.01 * pre2)
        o_r[...] = jnp.dot(h2, w3_r[...], preferred_element_type=jnp.float32) + b3_r[...]

    return pl.pallas_call(
        body,
        grid=(grid,),
        in_specs=[
            pl.BlockSpec((bn, df), lambda i: (i, 0)),
            pl.BlockSpec((2, bn, de), lambda i: (0, i, 0)),
            pl.BlockSpec((1, 1, bn), lambda i: (i, 0, 0)),
            pl.BlockSpec((dg, du), lambda i: (0, 0)),
            pl.BlockSpec((df, hh), lambda i: (0, 0)),
            pl.BlockSpec((de, hh), lambda i: (0, 0)),
            pl.BlockSpec((du, hh), lambda i: (0, 0)),
            pl.BlockSpec((1, hh), lambda i: (0, 0)),
            pl.BlockSpec((hh, hh), lambda i: (0, 0)),
            pl.BlockSpec((1, hh), lambda i: (0, 0)),
            pl.BlockSpec((hh, t), lambda i: (0, 0)),
            pl.BlockSpec((1, t), lambda i: (0, 0)),
        ],
        out_specs=pl.BlockSpec((bn, t), lambda i: (i, 0)),
        out_shape=jax.ShapeDtypeStruct((n, t), jnp.float32),
    )(x, parts, batch3, u, w1a, w1b, w1c, b1, w2, b2, w3, b3)


def kernel(x, edge_index, edge_attr, u, batch, W1, b1, W2, b2, W3, b3):
    n, df = x.shape
    e, de = edge_attr.shape
    ept = e // _NW
    ch = ept // _CK

    ei4 = edge_index.reshape(2, _NW, ch, _CK)
    n_pad = ((n + 8 * _NS - 1) // (8 * _NS)) * (8 * _NS)
    parts = _sc_scatter_partials(edge_attr, ei4, n_pad)

    w1a = W1[:df]
    w1b = W1[df:df + de]
    w1c = W1[df + de:]
    bn = 2000
    batch3 = batch.reshape(n // bn, 1, bn)
    return _tc_mlp(x, parts, batch3, u, w1a, w1b, w1c,
                   b1.reshape(1, -1), W2, b2.reshape(1, -1), W3,
                   b3.reshape(1, -1), bn)
